# baseline (speedup 1.0000x reference)
import functools

import jax
import jax.numpy as jnp
from jax import lax
from jax.experimental import pallas as pl
from jax.experimental.pallas import tpu as pltpu
from jax.experimental.pallas import tpu_sc as plsc

N = 10000
E = 320000
D = 128
NC = 2
NS = 16
NPAD = 10240
B = 80
EPT = E // NS
G = EPT // B
EPC = E // NS
SL = NPAD // NS
H = NPAD // NC
HG = H + 8
RS2 = H // NS
NB = 5
GRP = NB * B
CAP = 20480
GARB = 15000 << 14

_mesh = plsc.VectorSubcoreMesh(
    core_axis_name="c", subcore_axis_name="s", num_cores=NC, num_subcores=NS
)
_sc_params = pltpu.CompilerParams(needs_layout_passes=False)


def _rsqrt16(x):
    i = plsc.bitcast(x, jnp.int32)
    y = plsc.bitcast(jnp.int32(0x5F3759DF) - (i >> 1), jnp.float32)
    for _ in range(3):
        y = y * (1.5 - 0.5 * x * y * y)
    return y


@functools.partial(
    pl.kernel,
    out_type=(
        jax.ShapeDtypeStruct((NPAD,), jnp.float32),
        jax.ShapeDtypeStruct((NS, NPAD), jnp.float32),
        jax.ShapeDtypeStruct((NC, NS, CAP), jnp.int32),
        jax.ShapeDtypeStruct((NS, 16), jnp.int32),
    ),
    mesh=_mesh,
    compiler_params=_sc_params,
    scratch_types=[
        pltpu.VMEM((EPC,), jnp.int32),
        pltpu.VMEM((NPAD,), jnp.float32),
        pltpu.VMEM((CAP,), jnp.int32),
        pltpu.VMEM((CAP,), jnp.int32),
        pltpu.VMEM((16,), jnp.int32),
        pltpu.VMEM((SL,), jnp.float32),
        pltpu.VMEM((SL,), jnp.float32),
    ],
)
def _prep_kernel(pk_hbm, dinv_hbm, hists_sh, lists_hbm, counts_hbm,
                 pk_v, hist_v, l0_v, l1_v, cnt_v, tmp_v, acc_v):
    c = lax.axis_index("c")
    s = lax.axis_index("s")
    zero16 = jnp.zeros((16,), jnp.float32)
    ones16 = jnp.ones((16,), jnp.float32)
    garb16 = jnp.full((16,), GARB, jnp.int32)

    def zbody(i, _):
        hist_v[pl.ds(i * 16, 16)] = zero16
        return 0

    lax.fori_loop(0, NPAD // 16, zbody, 0)
    pltpu.sync_copy(pk_hbm.at[s], pk_v)

    def cbody(i, carry):
        off0, off1 = carry
        p = pk_v[pl.ds(i * 16, 16)]
        d = p >> 14
        plsc.addupdate_scatter(hist_v, [d], ones16)
        m0 = d < H
        plsc.store_compressed(l0_v.at[pl.ds(off0, 16)], p, mask=m0)
        plsc.store_compressed(l1_v.at[pl.ds(off1, 16)], p, mask=~m0)
        n0 = jnp.sum(jnp.where(m0, 1, 0))
        return off0 + n0, off1 + (16 - n0)

    off0, off1 = lax.fori_loop(0, EPC // 16, cbody,
                               (jnp.int32(0), jnp.int32(0)))
    pltpu.sync_copy(hist_v, hists_sh.at[s])

    def pad_list(lv, off):
        ngrp = jnp.maximum((off + (GRP - 1)) // GRP, 1)
        target = ngrp * GRP

        def fg(t, _):
            lv[pl.ds(off + t * 16, 16)] = garb16
            return 0

        lax.fori_loop(0, (target - off + 15) // 16, fg, 0)
        return ngrp

    ngrp0 = pad_list(l0_v, off0)
    ngrp1 = pad_list(l1_v, off1)
    lanes = lax.iota(jnp.int32, 16)
    cnt_v[pl.ds(0, 16)] = jnp.where(
        lanes == 0, ngrp0, jnp.where(lanes == 1, ngrp1, 0))

    @pl.when(c == 0)
    def _():
        pltpu.sync_copy(l0_v, lists_hbm.at[0, s])
        pltpu.sync_copy(l1_v, lists_hbm.at[1, s])
        pltpu.sync_copy(cnt_v, counts_hbm.at[s])

    plsc.subcore_barrier()

    def zacc(j, _):
        acc_v[pl.ds(j * 16, 16)] = zero16
        return 0

    lax.fori_loop(0, SL // 16, zacc, 0)

    def merge(t, _):
        pltpu.sync_copy(hists_sh.at[t, pl.ds(s * SL, SL)], tmp_v)

        def madd(j, _):
            acc_v[pl.ds(j * 16, 16)] = (acc_v[pl.ds(j * 16, 16)]
                                        + tmp_v[pl.ds(j * 16, 16)])
            return 0

        lax.fori_loop(0, SL // 16, madd, 0)
        return 0

    lax.fori_loop(0, NS, merge, 0)

    def rbody(j, _):
        d = acc_v[pl.ds(j * 16, 16)] + 1.0
        acc_v[pl.ds(j * 16, 16)] = _rsqrt16(d)
        return 0

    lax.fori_loop(0, SL // 16, rbody, 0)

    @pl.when(c == 0)
    def _():
        pltpu.sync_copy(acc_v, dinv_hbm.at[pl.ds(s * SL, SL)])


@functools.partial(
    pl.kernel,
    out_type=jax.ShapeDtypeStruct((NPAD, D), jnp.float32),
    mesh=_mesh,
    compiler_params=_sc_params,
    scratch_types=[
        pltpu.VMEM((CAP,), jnp.int32),
        pltpu.VMEM((16,), jnp.int32),
        pltpu.VMEM((NB, B), jnp.int32),
        pltpu.VMEM((NB, B), jnp.int32),
        pltpu.VMEM((NB, B, D), jnp.float32),
        pltpu.VMEM_SHARED((HG, D), jnp.float32),
    ] + [pltpu.SemaphoreType.DMA] * (2 * NB),
)
def _agg_kernel(hs_hbm, lists_hbm, counts_hbm, acc_hbm,
                pk_v, cnt_v, sidx_v, didx_v, rows_v, acc_sh, *sems):
    gsems = sems[:NB]
    ssems = sems[NB:]
    c = lax.axis_index("c")
    s = lax.axis_index("s")

    zero16 = jnp.zeros((16,), jnp.float32)

    def zbody(r, _):
        def zlane(l, _):
            rows_v[0, r, pl.ds(l * 16, 16)] = zero16
            return 0
        lax.fori_loop(0, D // 16, zlane, 0)
        return 0

    lax.fori_loop(0, B, zbody, 0)
    for k in range(RS2 // B):
        pltpu.sync_copy(rows_v.at[0], acc_sh.at[pl.ds(s * RS2 + k * B, B)])

    pltpu.sync_copy(lists_hbm.at[c, s], pk_v)
    pltpu.sync_copy(counts_hbm.at[s], cnt_v)
    lanes = lax.iota(jnp.int32, 16)
    ngrp = jnp.sum(jnp.where(lanes == c, cnt_v[pl.ds(0, 16)], 0))
    lo = c * H

    def unpack(g, buf):
        def lane(l, _):
            p = pk_v[pl.ds(g * B + l * 16, 16)]
            sidx_v[buf, pl.ds(l * 16, 16)] = p & 16383
            d = (p >> 14) - lo
            ok = (d >= 0) & (d < H)
            didx_v[buf, pl.ds(l * 16, 16)] = jnp.where(ok, d, H)
            return 0
        lax.fori_loop(0, B // 16, lane, 0)

    plsc.subcore_barrier()

    def fire_gather(buf, sem):
        pltpu.async_copy(hs_hbm.at[sidx_v.at[buf]], rows_v.at[buf], sem)

    def wait_gather(buf, sem):
        pltpu.make_async_copy(hs_hbm.at[sidx_v.at[buf]], rows_v.at[buf],
                              sem).wait()

    def fire_scatter(buf, sem):
        pltpu.async_copy(rows_v.at[buf], acc_sh.at[didx_v.at[buf]], sem,
                         add=True)

    def wait_scatter(buf, sem):
        pltpu.make_async_copy(rows_v.at[buf], acc_sh.at[didx_v.at[buf]],
                              sem).wait()

    for b in range(NB):
        unpack(b, b)
        fire_gather(b, gsems[b])

    def body(j, _):
        base = NB * j
        for b in range(NB):
            wait_gather(b, gsems[b])
            fire_scatter(b, ssems[b])
        for b in range(NB):
            wait_scatter(b, ssems[b])
            unpack(base + NB + b, b)
            fire_gather(b, gsems[b])
        return 0

    lax.fori_loop(0, ngrp - 1, body, 0)
    for b in range(NB):
        wait_gather(b, gsems[b])
        fire_scatter(b, ssems[b])
    for b in range(NB):
        wait_scatter(b, ssems[b])
    plsc.subcore_barrier()
    pltpu.sync_copy(acc_sh.at[pl.ds(s * RS2, RS2)],
                    acc_hbm.at[pl.ds(c * H + s * RS2, RS2)])


_BM = 1024


def _tc1_body(x_ref, w_ref, dinv_ref, out_ref):
    h = jnp.dot(x_ref[...], w_ref[...], preferred_element_type=jnp.float32)
    out_ref[...] = dinv_ref[...] * h


def _tc1(x_pad, w1, dinv2d):
    return pl.pallas_call(
        _tc1_body,
        grid=(NPAD // _BM,),
        in_specs=[
            pl.BlockSpec((_BM, D), lambda m: (m, 0)),
            pl.BlockSpec((D, D), lambda m: (0, 0)),
            pl.BlockSpec((_BM, 1), lambda m: (m, 0)),
        ],
        out_specs=pl.BlockSpec((_BM, D), lambda m: (m, 0)),
        out_shape=jax.ShapeDtypeStruct((NPAD, D), jnp.float32),
    )(x_pad, w1, dinv2d)


def _tc2_body(acc_ref, hs_ref, dinv_ref, b_ref, w_ref, out_ref):
    t = acc_ref[...] + hs_ref[...]
    h = jnp.maximum(dinv_ref[...] * t + b_ref[...], 0.0)
    h2 = jnp.dot(h, w_ref[...], preferred_element_type=jnp.float32)
    out_ref[...] = dinv_ref[...] * h2


def _tc2(acc1, hs1, dinv2d, b1r, w2):
    return pl.pallas_call(
        _tc2_body,
        grid=(NPAD // _BM,),
        in_specs=[
            pl.BlockSpec((_BM, D), lambda m: (m, 0)),
            pl.BlockSpec((_BM, D), lambda m: (m, 0)),
            pl.BlockSpec((_BM, 1), lambda m: (m, 0)),
            pl.BlockSpec((1, D), lambda m: (0, 0)),
            pl.BlockSpec((D, D), lambda m: (0, 0)),
        ],
        out_specs=pl.BlockSpec((_BM, D), lambda m: (m, 0)),
        out_shape=jax.ShapeDtypeStruct((NPAD, D), jnp.float32),
    )(acc1, hs1, dinv2d, b1r, w2)


def _tc3_body(acc_ref, hs_ref, dinv_ref, b_ref, out_ref):
    t = acc_ref[...] + hs_ref[...]
    logits = dinv_ref[...] * t + b_ref[...]
    m = jnp.max(logits, axis=1, keepdims=True)
    lse = jnp.log(jnp.sum(jnp.exp(logits - m), axis=1, keepdims=True)) + m
    out_ref[...] = logits - lse


def _tc3(acc2, hs2, dinv2d, b2r):
    return pl.pallas_call(
        _tc3_body,
        grid=(NPAD // _BM,),
        in_specs=[
            pl.BlockSpec((_BM, D), lambda m: (m, 0)),
            pl.BlockSpec((_BM, D), lambda m: (m, 0)),
            pl.BlockSpec((_BM, 1), lambda m: (m, 0)),
            pl.BlockSpec((1, D), lambda m: (0, 0)),
        ],
        out_specs=pl.BlockSpec((_BM, D), lambda m: (m, 0)),
        out_shape=jax.ShapeDtypeStruct((NPAD, D), jnp.float32),
    )(acc2, hs2, dinv2d, b2r)


def kernel(x, edge_index, W1, b1, W2, b2):
    src = edge_index[0].astype(jnp.int32)
    dst = edge_index[1].astype(jnp.int32)
    x_pad = jnp.pad(x, ((0, NPAD - N), (0, 0)))
    packed2d = (src | (dst << 14)).reshape(NS, EPC)

    dinv, _hists, lists, counts = _prep_kernel(packed2d)
    dinv2d = dinv[:, None]
    hs1 = _tc1(x_pad, W1, dinv2d)
    acc1 = _agg_kernel(hs1, lists, counts)
    hs2 = _tc2(acc1, hs1, dinv2d, b1[None, :], W2)
    acc2 = _agg_kernel(hs2, lists, counts)
    out = _tc3(acc2, hs2, dinv2d, b2[None, :])
    return out[:N]


# speedup vs baseline: 1.3101x; 1.3101x over previous
import functools

import jax
import jax.numpy as jnp
from jax import lax
from jax.experimental import pallas as pl
from jax.experimental.pallas import tpu as pltpu
from jax.experimental.pallas import tpu_sc as plsc

N = 10000
E = 320000
D = 128
NC = 2
NS = 16
NPAD = 10240
B = 80
EPT = E // NS
G = EPT // B
EPC = E // NS
SL = NPAD // NS
H = NPAD // NC
HG = H + 8
RS2 = H // NS
NB = 5

_mesh = plsc.VectorSubcoreMesh(
    core_axis_name="c", subcore_axis_name="s", num_cores=NC, num_subcores=NS
)
_sc_params = pltpu.CompilerParams(needs_layout_passes=False)


def _rsqrt16(x):
    i = plsc.bitcast(x, jnp.int32)
    y = plsc.bitcast(jnp.int32(0x5F3759DF) - (i >> 1), jnp.float32)
    for _ in range(3):
        y = y * (1.5 - 0.5 * x * y * y)
    return y


@functools.partial(
    pl.kernel,
    out_type=(
        jax.ShapeDtypeStruct((NPAD,), jnp.float32),
        jax.ShapeDtypeStruct((NS, NPAD), jnp.float32),
    ),
    mesh=_mesh,
    compiler_params=_sc_params,
    scratch_types=[
        pltpu.VMEM((EPC,), jnp.int32),
        pltpu.VMEM((NPAD,), jnp.float32),
        pltpu.VMEM((SL,), jnp.float32),
        pltpu.VMEM((SL,), jnp.float32),
    ],
)
def _dinv_kernel(dst_hbm, dinv_hbm, hists_sh, dst_v, hist_v, tmp_v, acc_v):
    c = lax.axis_index("c")
    s = lax.axis_index("s")
    zero16 = jnp.zeros((16,), jnp.float32)
    ones16 = jnp.ones((16,), jnp.float32)

    def zbody(i, _):
        hist_v[pl.ds(i * 16, 16)] = zero16
        return 0

    lax.fori_loop(0, NPAD // 16, zbody, 0)
    pltpu.sync_copy(dst_hbm.at[pl.ds(s * EPC, EPC)], dst_v)

    def cbody(i, _):
        idx = dst_v[pl.ds(i * 16, 16)]
        plsc.addupdate_scatter(hist_v, [idx], ones16)
        return 0

    lax.fori_loop(0, EPC // 16, cbody, 0)
    pltpu.sync_copy(hist_v, hists_sh.at[s])
    plsc.subcore_barrier()

    def zacc(j, _):
        acc_v[pl.ds(j * 16, 16)] = zero16
        return 0

    lax.fori_loop(0, SL // 16, zacc, 0)

    def merge(t, _):
        pltpu.sync_copy(hists_sh.at[t, pl.ds(s * SL, SL)], tmp_v)

        def madd(j, _):
            acc_v[pl.ds(j * 16, 16)] = (acc_v[pl.ds(j * 16, 16)]
                                        + tmp_v[pl.ds(j * 16, 16)])
            return 0

        lax.fori_loop(0, SL // 16, madd, 0)
        return 0

    lax.fori_loop(0, NS, merge, 0)

    def rbody(j, _):
        d = acc_v[pl.ds(j * 16, 16)] + 1.0
        acc_v[pl.ds(j * 16, 16)] = _rsqrt16(d)
        return 0

    lax.fori_loop(0, SL // 16, rbody, 0)

    @pl.when(c == 0)
    def _():
        pltpu.sync_copy(acc_v, dinv_hbm.at[pl.ds(s * SL, SL)])


@functools.partial(
    pl.kernel,
    out_type=jax.ShapeDtypeStruct((NPAD, D), jnp.float32),
    mesh=_mesh,
    compiler_params=_sc_params,
    scratch_types=[
        pltpu.VMEM((G, B), jnp.int32),
        pltpu.VMEM((NB, B), jnp.int32),
        pltpu.VMEM((NB, B), jnp.int32),
        pltpu.VMEM((NB, B, D), jnp.float32),
        pltpu.VMEM_SHARED((HG, D), jnp.float32),
    ] + [pltpu.SemaphoreType.DMA] * (2 * NB),
)
def _agg_kernel(hs_hbm, pk_hbm, acc_hbm,
                pk_v, sidx_v, didx_v, rows_v, acc_sh, *sems):
    gsems = sems[:NB]
    ssems = sems[NB:]
    c = lax.axis_index("c")
    s = lax.axis_index("s")

    pltpu.sync_copy(pk_hbm.at[s], pk_v)
    lo = c * H

    def unpack(g, buf):
        def lane(l, _):
            p = pk_v[g, pl.ds(l * 16, 16)]
            sidx_v[buf, pl.ds(l * 16, 16)] = p & 16383
            d = (p >> 14) - lo
            ok = (d >= 0) & (d < H)
            didx_v[buf, pl.ds(l * 16, 16)] = jnp.where(ok, d, H)
            return 0
        lax.fori_loop(0, B // 16, lane, 0)

    def fire_gather(buf, sem):
        pltpu.async_copy(hs_hbm.at[sidx_v.at[buf]], rows_v.at[buf], sem)

    def wait_gather(buf, sem):
        pltpu.make_async_copy(hs_hbm.at[sidx_v.at[buf]], rows_v.at[buf],
                              sem).wait()

    def fire_scatter(buf, sem):
        pltpu.async_copy(rows_v.at[buf], acc_sh.at[didx_v.at[buf]], sem,
                         add=True)

    def wait_scatter(buf, sem):
        pltpu.make_async_copy(rows_v.at[buf], acc_sh.at[didx_v.at[buf]],
                              sem).wait()

    for b in range(1, NB):
        unpack(b, b)
        fire_gather(b, gsems[b])

    zero16 = jnp.zeros((16,), jnp.float32)

    def zbody(r, _):
        def zlane(l, _):
            rows_v[0, r, pl.ds(l * 16, 16)] = zero16
            return 0
        lax.fori_loop(0, D // 16, zlane, 0)
        return 0

    lax.fori_loop(0, B, zbody, 0)
    for k in range(RS2 // B):
        pltpu.sync_copy(rows_v.at[0], acc_sh.at[pl.ds(s * RS2 + k * B, B)])

    unpack(0, 0)
    fire_gather(0, gsems[0])
    plsc.subcore_barrier()

    NR = G // NB

    def body(j, _):
        base = NB * j
        for b in range(NB):
            wait_gather(b, gsems[b])
            fire_scatter(b, ssems[b])

        @pl.when(j < NR - 1)
        def _():
            for b in range(NB):
                wait_scatter(b, ssems[b])
                unpack(base + NB + b, b)
                fire_gather(b, gsems[b])

        return 0

    lax.fori_loop(0, NR, body, 0)
    for b in range(NB):
        wait_scatter(b, ssems[b])
    plsc.subcore_barrier()
    pltpu.sync_copy(acc_sh.at[pl.ds(s * RS2, RS2)],
                    acc_hbm.at[pl.ds(c * H + s * RS2, RS2)])


_BM = 1024


def _tc1_body(x_ref, w_ref, dinv_ref, out_ref):
    h = jnp.dot(x_ref[...], w_ref[...], preferred_element_type=jnp.float32)
    out_ref[...] = dinv_ref[...] * h


def _tc1(x_pad, w1, dinv2d):
    return pl.pallas_call(
        _tc1_body,
        grid=(NPAD // _BM,),
        in_specs=[
            pl.BlockSpec((_BM, D), lambda m: (m, 0)),
            pl.BlockSpec((D, D), lambda m: (0, 0)),
            pl.BlockSpec((_BM, 1), lambda m: (m, 0)),
        ],
        out_specs=pl.BlockSpec((_BM, D), lambda m: (m, 0)),
        out_shape=jax.ShapeDtypeStruct((NPAD, D), jnp.float32),
    )(x_pad, w1, dinv2d)


def _tc2_body(acc_ref, hs_ref, dinv_ref, b_ref, w_ref, out_ref):
    t = acc_ref[...] + hs_ref[...]
    h = jnp.maximum(dinv_ref[...] * t + b_ref[...], 0.0)
    h2 = jnp.dot(h, w_ref[...], preferred_element_type=jnp.float32)
    out_ref[...] = dinv_ref[...] * h2


def _tc2(acc1, hs1, dinv2d, b1r, w2):
    return pl.pallas_call(
        _tc2_body,
        grid=(NPAD // _BM,),
        in_specs=[
            pl.BlockSpec((_BM, D), lambda m: (m, 0)),
            pl.BlockSpec((_BM, D), lambda m: (m, 0)),
            pl.BlockSpec((_BM, 1), lambda m: (m, 0)),
            pl.BlockSpec((1, D), lambda m: (0, 0)),
            pl.BlockSpec((D, D), lambda m: (0, 0)),
        ],
        out_specs=pl.BlockSpec((_BM, D), lambda m: (m, 0)),
        out_shape=jax.ShapeDtypeStruct((NPAD, D), jnp.float32),
    )(acc1, hs1, dinv2d, b1r, w2)


def _tc3_body(acc_ref, hs_ref, dinv_ref, b_ref, out_ref):
    t = acc_ref[...] + hs_ref[...]
    logits = dinv_ref[...] * t + b_ref[...]
    m = jnp.max(logits, axis=1, keepdims=True)
    lse = jnp.log(jnp.sum(jnp.exp(logits - m), axis=1, keepdims=True)) + m
    out_ref[...] = logits - lse


def _tc3(acc2, hs2, dinv2d, b2r):
    return pl.pallas_call(
        _tc3_body,
        grid=(NPAD // _BM,),
        in_specs=[
            pl.BlockSpec((_BM, D), lambda m: (m, 0)),
            pl.BlockSpec((_BM, D), lambda m: (m, 0)),
            pl.BlockSpec((_BM, 1), lambda m: (m, 0)),
            pl.BlockSpec((1, D), lambda m: (0, 0)),
        ],
        out_specs=pl.BlockSpec((_BM, D), lambda m: (m, 0)),
        out_shape=jax.ShapeDtypeStruct((NPAD, D), jnp.float32),
    )(acc2, hs2, dinv2d, b2r)


def kernel(x, edge_index, W1, b1, W2, b2):
    src = edge_index[0].astype(jnp.int32)
    dst = edge_index[1].astype(jnp.int32)
    x_pad = jnp.pad(x, ((0, NPAD - N), (0, 0)))
    packed3d = (src | (dst << 14)).reshape(NS, G, B)

    dinv, _hists = _dinv_kernel(dst)
    dinv2d = dinv[:, None]
    hs1 = _tc1(x_pad, W1, dinv2d)
    acc1 = _agg_kernel(hs1, packed3d)
    hs2 = _tc2(acc1, hs1, dinv2d, b1[None, :], W2)
    acc2 = _agg_kernel(hs2, packed3d)
    out = _tc3(acc2, hs2, dinv2d, b2[None, :])
    return out[:N]
